# Initial kernel scaffold; baseline (speedup 1.0000x reference)
import jax
import jax.numpy as jnp
from jax.experimental import pallas as pl


def kernel(inputs, token_table, pos_table):
    raise NotImplementedError("write your pallas kernel here")


# speedup vs baseline: 3.9528x; 3.9528x over previous
import functools

import jax
import jax.numpy as jnp
from jax import lax
from jax.experimental import pallas as pl
from jax.experimental.pallas import tpu as pltpu
from jax.experimental.pallas import tpu_sc as plsc

_NUM_WORKERS = 32


def kernel(inputs, token_table, pos_table):
    B, S = inputs.shape
    V, D = token_table.shape
    seq_per_w = B // _NUM_WORKERS
    half = S // 2

    idx3 = inputs.astype(jnp.int32).reshape(B, 2, half)

    mesh = plsc.VectorSubcoreMesh(core_axis_name="c", subcore_axis_name="s")

    @functools.partial(
        pl.kernel,
        mesh=mesh,
        out_type=jax.ShapeDtypeStruct((B, S, D), jnp.float32),
        scratch_types=[
            pltpu.VMEM((2, half), jnp.int32),
            pltpu.VMEM((S, D), jnp.float32),
            pltpu.VMEM((S, D), jnp.float32),
            pltpu.SemaphoreType.DMA,
        ],
    )
    def emb_kernel(idx_hbm, tok_hbm, pos_hbm, out_hbm, idx_v, rows_v, pos_v, sem):
        wid = lax.axis_index("s") * 2 + lax.axis_index("c")
        pltpu.sync_copy(pos_hbm, pos_v)

        def per_seq(j, carry):
            b = wid * seq_per_w + j
            pltpu.sync_copy(idx_hbm.at[b], idx_v)
            cp0 = pltpu.async_copy(
                tok_hbm.at[idx_v.at[0]], rows_v.at[pl.ds(0, half)], sem
            )
            cp1 = pltpu.async_copy(
                tok_hbm.at[idx_v.at[1]], rows_v.at[pl.ds(half, half)], sem
            )
            cp0.wait()
            cp1.wait()

            def add_row(r, c2):
                for c in range(D // 16):
                    sl = pl.ds(c * 16, 16)
                    rows_v[r, sl] = rows_v[r, sl] + pos_v[r, sl]
                return c2

            lax.fori_loop(0, S, add_row, 0)
            pltpu.sync_copy(rows_v, out_hbm.at[b])
            return carry

        lax.fori_loop(0, seq_per_w, per_seq, 0)

    return emb_kernel(idx3, token_table, pos_table)


# speedup vs baseline: 6.2167x; 1.5727x over previous
import functools

import jax
import jax.numpy as jnp
from jax import lax
from jax.experimental import pallas as pl
from jax.experimental.pallas import tpu as pltpu
from jax.experimental.pallas import tpu_sc as plsc

_NUM_WORKERS = 32


def kernel(inputs, token_table, pos_table):
    B, S = inputs.shape
    V, D = token_table.shape
    seq_per_w = B // _NUM_WORKERS
    half = S // 2

    idx3 = inputs.astype(jnp.int32).reshape(B, 2, half)

    mesh = plsc.VectorSubcoreMesh(core_axis_name="c", subcore_axis_name="s")

    @functools.partial(
        pl.kernel,
        mesh=mesh,
        out_type=jax.ShapeDtypeStruct((B, S, D), jnp.float32),
        scratch_types=[
            pltpu.VMEM((seq_per_w, 2, half), jnp.int32),
            pltpu.VMEM((S, D), jnp.float32),
            pltpu.VMEM((S, D), jnp.float32),
            pltpu.VMEM((S, D), jnp.float32),
            pltpu.SemaphoreType.DMA,
            pltpu.SemaphoreType.DMA,
            pltpu.SemaphoreType.DMA,
            pltpu.SemaphoreType.DMA,
        ],
    )
    def emb_kernel(idx_hbm, tok_hbm, pos_hbm, out_hbm, idx_v, pos_v,
                   rows0, rows1, gsem0, gsem1, ssem0, ssem1):
        wid = lax.axis_index("s") * 2 + lax.axis_index("c")
        b0 = wid * seq_per_w
        pltpu.sync_copy(idx_hbm.at[pl.ds(b0, seq_per_w)], idx_v)
        pltpu.sync_copy(pos_hbm, pos_v)

        def issue_gather(j, buf, sem):
            pltpu.async_copy(tok_hbm.at[idx_v.at[j, 0]],
                             buf.at[pl.ds(0, half)], sem)
            pltpu.async_copy(tok_hbm.at[idx_v.at[j, 1]],
                             buf.at[pl.ds(half, half)], sem)

        def wait_gather(buf, sem):
            pltpu.make_async_copy(tok_hbm.at[idx_v.at[0, 0]],
                                  buf.at[pl.ds(0, half)], sem).wait()
            pltpu.make_async_copy(tok_hbm.at[idx_v.at[0, 1]],
                                  buf.at[pl.ds(half, half)], sem).wait()

        def drain_store(buf, sem):
            pltpu.make_async_copy(buf, out_hbm.at[0], sem).wait()

        def add_pos(buf):
            def body(i, c):
                for k in range(8):
                    r = i * 8 + k
                    for c8 in range(D // 16):
                        sl = pl.ds(c8 * 16, 16)
                        buf[r, sl] = buf[r, sl] + pos_v[r, sl]
                return c
            lax.fori_loop(0, S // 8, body, 0)

        issue_gather(0, rows0, gsem0)

        def step(jj, carry):
            j0 = jj * 2
            pl.when(jj >= 1)(lambda: drain_store(rows1, ssem1))
            issue_gather(j0 + 1, rows1, gsem1)
            wait_gather(rows0, gsem0)
            add_pos(rows0)
            pltpu.async_copy(rows0, out_hbm.at[b0 + j0], ssem0)
            def prefetch_a():
                drain_store(rows0, ssem0)
                issue_gather(j0 + 2, rows0, gsem0)
            pl.when(jj < seq_per_w // 2 - 1)(prefetch_a)
            wait_gather(rows1, gsem1)
            add_pos(rows1)
            pltpu.async_copy(rows1, out_hbm.at[b0 + j0 + 1], ssem1)
            return carry

        lax.fori_loop(0, seq_per_w // 2, step, 0)
        drain_store(rows0, ssem0)
        drain_store(rows1, ssem1)

    return emb_kernel(idx3, token_table, pos_table)


# speedup vs baseline: 6.3659x; 1.0240x over previous
import functools

import jax
import jax.numpy as jnp
from jax import lax
from jax.experimental import pallas as pl
from jax.experimental.pallas import tpu as pltpu
from jax.experimental.pallas import tpu_sc as plsc

_NUM_WORKERS = 32


def kernel(inputs, token_table, pos_table):
    B, S = inputs.shape
    V, D = token_table.shape
    seq_per_w = B // _NUM_WORKERS
    half = S // 2

    idx3 = inputs.astype(jnp.int32).reshape(B, 2, half)

    mesh = plsc.VectorSubcoreMesh(core_axis_name="c", subcore_axis_name="s")

    @functools.partial(
        pl.kernel,
        mesh=mesh,
        out_type=jax.ShapeDtypeStruct((B, S, D), jnp.float32),
        scratch_types=[
            pltpu.VMEM((seq_per_w, 2, half), jnp.int32),
            pltpu.VMEM((S, D), jnp.float32),
            pltpu.VMEM((S, D), jnp.float32),
            pltpu.VMEM((S, D), jnp.float32),
            pltpu.SemaphoreType.DMA,
            pltpu.SemaphoreType.DMA,
            pltpu.SemaphoreType.DMA,
            pltpu.SemaphoreType.DMA,
        ],
    )
    def emb_kernel(idx_hbm, tok_hbm, pos_hbm, out_hbm, idx_v, pos_v,
                   rows0, rows1, gsem0, gsem1, ssem0, ssem1):
        wid = lax.axis_index("s") * 2 + lax.axis_index("c")
        b0 = wid * seq_per_w
        pltpu.sync_copy(idx_hbm.at[pl.ds(b0, seq_per_w)], idx_v)
        pltpu.sync_copy(pos_hbm, pos_v)

        def issue_gather(j, buf, sem):
            pltpu.async_copy(tok_hbm.at[idx_v.at[j, 0]],
                             buf.at[pl.ds(0, half)], sem)
            pltpu.async_copy(tok_hbm.at[idx_v.at[j, 1]],
                             buf.at[pl.ds(half, half)], sem)

        def wait_gather(buf, sem):
            pltpu.make_async_copy(tok_hbm.at[idx_v.at[0, 0]],
                                  buf.at[pl.ds(0, half)], sem).wait()
            pltpu.make_async_copy(tok_hbm.at[idx_v.at[0, 1]],
                                  buf.at[pl.ds(half, half)], sem).wait()

        def drain_store(buf, sem):
            pltpu.make_async_copy(buf, out_hbm.at[0], sem).wait()

        def add_pos(buf):
            def body(i, c):
                for k in range(8):
                    r = i * 8 + k
                    for c8 in range(D // 16):
                        sl = pl.ds(c8 * 16, 16)
                        plsc.addupdate(buf.at[r, sl], pos_v[r, sl])
                return c
            lax.fori_loop(0, S // 8, body, 0)

        issue_gather(0, rows0, gsem0)

        def step(jj, carry):
            j0 = jj * 2
            pl.when(jj >= 1)(lambda: drain_store(rows1, ssem1))
            issue_gather(j0 + 1, rows1, gsem1)
            wait_gather(rows0, gsem0)
            add_pos(rows0)
            pltpu.async_copy(rows0, out_hbm.at[b0 + j0], ssem0)
            def prefetch_a():
                drain_store(rows0, ssem0)
                issue_gather(j0 + 2, rows0, gsem0)
            pl.when(jj < seq_per_w // 2 - 1)(prefetch_a)
            wait_gather(rows1, gsem1)
            add_pos(rows1)
            pltpu.async_copy(rows1, out_hbm.at[b0 + j0 + 1], ssem1)
            return carry

        lax.fori_loop(0, seq_per_w // 2, step, 0)
        drain_store(rows0, ssem0)
        drain_store(rows1, ssem1)

    return emb_kernel(idx3, token_table, pos_table)


# speedup vs baseline: 7.4581x; 1.1716x over previous
import functools

import jax
import jax.numpy as jnp
from jax import lax
from jax.experimental import pallas as pl
from jax.experimental.pallas import tpu as pltpu
from jax.experimental.pallas import tpu_sc as plsc

_NUM_WORKERS = 32


def kernel(inputs, token_table, pos_table):
    B, S = inputs.shape
    V, D = token_table.shape
    seq_per_w = B // _NUM_WORKERS
    n_units = seq_per_w * 2
    sizes = (96, S - 96)
    offs = (0, 96)
    bufsz = max(sizes)

    idx1 = inputs.astype(jnp.int32).reshape(-1)

    mesh = plsc.VectorSubcoreMesh(core_axis_name="c", subcore_axis_name="s")

    @functools.partial(
        pl.kernel,
        mesh=mesh,
        out_type=jax.ShapeDtypeStruct((B, S, D), jnp.float32),
        scratch_types=[
            pltpu.VMEM((seq_per_w * S,), jnp.int32),
            pltpu.VMEM((S, D), jnp.float32),
            pltpu.VMEM((bufsz, D), jnp.float32),
            pltpu.VMEM((bufsz, D), jnp.float32),
            pltpu.VMEM((bufsz, D), jnp.float32),
            pltpu.VMEM((bufsz, D), jnp.float32),
            pltpu.SemaphoreType.DMA,
            pltpu.SemaphoreType.DMA,
            pltpu.SemaphoreType.DMA,
            pltpu.SemaphoreType.DMA,
            pltpu.SemaphoreType.DMA,
            pltpu.SemaphoreType.DMA,
            pltpu.SemaphoreType.DMA,
            pltpu.SemaphoreType.DMA,
        ],
    )
    def emb_kernel(idx_hbm, tok_hbm, pos_hbm, out_hbm, idx_v, pos_v,
                   b0v, b1v, b2v, b3v, g0, g1, g2, g3, s0, s1, s2, s3):
        bufs = (b0v, b1v, b2v, b3v)
        gsems = (g0, g1, g2, g3)
        ssems = (s0, s1, s2, s3)
        wid = lax.axis_index("s") * 2 + lax.axis_index("c")
        seq0 = wid * seq_per_w
        pltpu.sync_copy(idx_hbm.at[pl.ds(seq0 * S, seq_per_w * S)], idx_v)
        pltpu.sync_copy(pos_hbm, pos_v)

        def issue_gather(u, p, r):
            pltpu.async_copy(
                tok_hbm.at[idx_v.at[pl.ds((u // 2) * S + offs[p], sizes[p])]],
                bufs[r].at[pl.ds(0, sizes[p])], gsems[r])

        def wait_gather(p, r):
            pltpu.make_async_copy(
                tok_hbm.at[idx_v.at[pl.ds(offs[p], sizes[p])]],
                bufs[r].at[pl.ds(0, sizes[p])], gsems[r]).wait()

        def issue_store(u, p, r):
            pltpu.async_copy(
                bufs[r].at[pl.ds(0, sizes[p])],
                out_hbm.at[seq0 + u // 2, pl.ds(offs[p], sizes[p])],
                ssems[r])

        def drain_store(p, r):
            pltpu.make_async_copy(
                bufs[r].at[pl.ds(0, sizes[p])],
                out_hbm.at[0, pl.ds(offs[p], sizes[p])], ssems[r]).wait()

        def add_pos(p, r):
            buf = bufs[r]

            def body(i, c):
                for k in range(4):
                    rr = i * 4 + k
                    for c8 in range(D // 16):
                        sl = pl.ds(c8 * 16, 16)
                        plsc.addupdate(buf.at[rr, sl],
                                       pos_v[offs[p] + rr, sl])
                return c
            lax.fori_loop(0, sizes[p] // 4, body, 0)

        issue_gather(0, 0, 0)
        issue_gather(1, 1, 1)

        def step(t, carry):
            for k in range(4):
                u = t * 4 + k
                p = k % 2
                r = k
                rn = (k + 2) % 4
                if k < 2:
                    pl.when(t >= 1)(lambda p2=p, r2=rn: drain_store(p2, r2))
                else:
                    drain_store(p, rn)
                if k < 2:
                    issue_gather(u + 2, p, rn)
                else:
                    pl.when(t < n_units // 4 - 1)(
                        lambda u2=u, p2=p, r2=rn: issue_gather(u2 + 2, p2, r2))
                wait_gather(p, r)
                add_pos(p, r)
                issue_store(u, p, r)
            return carry

        lax.fori_loop(0, n_units // 4, step, 0)
        drain_store(0, 2)
        drain_store(1, 3)

    return emb_kernel(idx1, token_table, pos_table)
